# baseline (speedup 1.0000x reference)
import functools

import jax
import jax.numpy as jnp
from jax.experimental import pallas as pl
from jax.experimental.pallas import tpu as pltpu

B, S, H = 2, 2048, 1024
E, K, F = 8, 2, 256
SHARED_F = 512
T = B * S
GU = 2 * F
W1_N = E * GU + 2 * SHARED_F
W2_K = E * F + SHARED_F


def _moe_block(x_ref, gwt_ref, w1_ref, w2_ref, out_ref):
    x32 = x_ref[...]
    bt = x32.shape[0]

    logits = jnp.dot(x32, gwt_ref[...], preferred_element_type=jnp.float32)
    m = jnp.max(logits, axis=-1, keepdims=True)
    p = jnp.exp(logits - m)
    scores = p / jnp.sum(p, axis=-1, keepdims=True)
    lane = jax.lax.broadcasted_iota(jnp.int32, (bt, E), 1)
    i1 = jnp.argmax(scores, axis=-1)[:, None]
    m1 = jnp.max(scores, axis=-1, keepdims=True)
    masked = jnp.where(lane == i1, -1.0, scores)
    i2 = jnp.argmax(masked, axis=-1)[:, None]
    m2 = jnp.max(masked, axis=-1, keepdims=True)
    c = jnp.where(lane == i1, m1, 0.0) + jnp.where(lane == i2, m2, 0.0)

    xb = x32.astype(jnp.bfloat16)

    gu = jnp.dot(xb, w1_ref[...], preferred_element_type=jnp.float32)

    parts = []
    for e in range(E):
        g = gu[:, e * GU:e * GU + F]
        u = gu[:, e * GU + F:(e + 1) * GU]
        parts.append(c[:, e:e + 1] * (g * jax.nn.sigmoid(g) * u))
    sg = gu[:, E * GU:E * GU + SHARED_F]
    su = gu[:, E * GU + SHARED_F:]
    parts.append(sg * jax.nn.sigmoid(sg) * su)
    inter = jnp.concatenate(parts, axis=1).astype(jnp.bfloat16)

    out_ref[...] = jnp.dot(inter, w2_ref[...],
                           preferred_element_type=jnp.float32)


@functools.partial(jax.jit, static_argnames=("bt",))
def _moe(x, gwt, w1, w2, bt=512):
    grid = (T // bt,)
    return pl.pallas_call(
        _moe_block,
        grid=grid,
        in_specs=[
            pl.BlockSpec((bt, H), lambda i: (i, 0)),
            pl.BlockSpec((H, E), lambda i: (0, 0)),
            pl.BlockSpec((H, W1_N), lambda i: (0, 0)),
            pl.BlockSpec((W2_K, H), lambda i: (0, 0)),
        ],
        out_specs=pl.BlockSpec((bt, H), lambda i: (i, 0)),
        out_shape=jax.ShapeDtypeStruct((T, H), jnp.float32),
    )(x, gwt, w1, w2)


def kernel(hidden_states, gate_w, w_gate, w_up, w_down, ws_gate, ws_up,
           ws_down):
    x = hidden_states.reshape(-1, H)
    wgu = jnp.concatenate(
        [w_gate.astype(jnp.bfloat16), w_up.astype(jnp.bfloat16)],
        axis=2)
    w1 = jnp.concatenate(
        [wgu.transpose(1, 0, 2).reshape(H, E * GU),
         ws_gate.astype(jnp.bfloat16), ws_up.astype(jnp.bfloat16)],
        axis=1)
    w2 = jnp.concatenate(
        [w_down.astype(jnp.bfloat16).reshape(E * F, H),
         ws_down.astype(jnp.bfloat16)], axis=0)
    out = _moe(x, gate_w.T.astype(jnp.float32), w1, w2)
    return out.reshape(B, S, H)


# speedup vs baseline: 1.3799x; 1.3799x over previous
import functools

import jax
import jax.numpy as jnp
from jax.experimental import pallas as pl
from jax.experimental.pallas import tpu as pltpu

B, S, H = 2, 2048, 1024
E, K, F = 8, 2, 256
SHARED_F = 512
T = B * S
GU = 2 * F
W1_N = E * GU + 2 * SHARED_F
W2_K = E * F + SHARED_F


def _moe_block(x_ref, gwt_ref, wgu_ref, wsgu_ref, w2_ref, out_ref):
    x32 = x_ref[...]
    bt = x32.shape[0]

    logits = jnp.dot(x32, gwt_ref[...], preferred_element_type=jnp.float32)
    m = jnp.max(logits, axis=-1, keepdims=True)
    p = jnp.exp(logits - m)
    scores = p / jnp.sum(p, axis=-1, keepdims=True)
    lane = jax.lax.broadcasted_iota(jnp.int32, (bt, E), 1)
    i1 = jnp.argmax(scores, axis=-1)[:, None]
    m1 = jnp.max(scores, axis=-1, keepdims=True)
    masked = jnp.where(lane == i1, -1.0, scores)
    i2 = jnp.argmax(masked, axis=-1)[:, None]
    m2 = jnp.max(masked, axis=-1, keepdims=True)
    c = jnp.where(lane == i1, m1, 0.0) + jnp.where(lane == i2, m2, 0.0)

    xb = x32.astype(jnp.bfloat16)

    parts = []
    for e in range(E):
        gu = jnp.dot(xb, wgu_ref[e], preferred_element_type=jnp.float32)
        g, u = gu[:, :F], gu[:, F:]
        parts.append(c[:, e:e + 1] * (g * jax.nn.sigmoid(g) * u))
    gus = jnp.dot(xb, wsgu_ref[...], preferred_element_type=jnp.float32)
    sg, su = gus[:, :SHARED_F], gus[:, SHARED_F:]
    parts.append(sg * jax.nn.sigmoid(sg) * su)
    inter = jnp.concatenate(parts, axis=1).astype(jnp.bfloat16)

    out_ref[...] = jnp.dot(inter, w2_ref[...],
                           preferred_element_type=jnp.float32)


@functools.partial(jax.jit, static_argnames=("bt",))
def _moe(x, gwt, wgu, wsgu, w2, bt=512):
    grid = (T // bt,)
    return pl.pallas_call(
        _moe_block,
        grid=grid,
        in_specs=[
            pl.BlockSpec((bt, H), lambda i: (i, 0)),
            pl.BlockSpec((H, E), lambda i: (0, 0)),
            pl.BlockSpec((E, H, GU), lambda i: (0, 0, 0)),
            pl.BlockSpec((H, 2 * SHARED_F), lambda i: (0, 0)),
            pl.BlockSpec((W2_K, H), lambda i: (0, 0)),
        ],
        out_specs=pl.BlockSpec((bt, H), lambda i: (i, 0)),
        out_shape=jax.ShapeDtypeStruct((T, H), jnp.float32),
    )(x, gwt, wgu, wsgu, w2)


def kernel(hidden_states, gate_w, w_gate, w_up, w_down, ws_gate, ws_up,
           ws_down):
    x = hidden_states.reshape(-1, H)
    wgu = jnp.concatenate(
        [w_gate.astype(jnp.bfloat16), w_up.astype(jnp.bfloat16)],
        axis=2)
    wsgu = jnp.concatenate(
        [ws_gate.astype(jnp.bfloat16), ws_up.astype(jnp.bfloat16)],
        axis=1)
    w2 = jnp.concatenate(
        [w_down.astype(jnp.bfloat16).reshape(E * F, H),
         ws_down.astype(jnp.bfloat16)], axis=0)
    out = _moe(x, gate_w.T.astype(jnp.float32), wgu, wsgu, w2)
    return out.reshape(B, S, H)


# speedup vs baseline: 1.4359x; 1.0406x over previous
import functools

import jax
import jax.numpy as jnp
from jax.experimental import pallas as pl
from jax.experimental.pallas import tpu as pltpu

B, S, H = 2, 2048, 1024
E, K, F = 8, 2, 256
SHARED_F = 512
T = B * S


def _moe_block(x_ref, gwt_ref, wg_ref, wu_ref, wd_ref, wsg_ref, wsu_ref,
               wsd_ref, out_ref):
    x32 = x_ref[...]
    bt = x32.shape[0]

    logits = jnp.dot(x32, gwt_ref[...], preferred_element_type=jnp.float32)
    m = jnp.max(logits, axis=-1, keepdims=True)
    p = jnp.exp(logits - m)
    scores = p / jnp.sum(p, axis=-1, keepdims=True)
    lane = jax.lax.broadcasted_iota(jnp.int32, (bt, E), 1)
    i1 = jnp.argmax(scores, axis=-1)[:, None]
    m1 = jnp.max(scores, axis=-1, keepdims=True)
    masked = jnp.where(lane == i1, -1.0, scores)
    i2 = jnp.argmax(masked, axis=-1)[:, None]
    m2 = jnp.max(masked, axis=-1, keepdims=True)
    c = jnp.where(lane == i1, m1, 0.0) + jnp.where(lane == i2, m2, 0.0)

    sg = jnp.dot(x32, wsg_ref[...], preferred_element_type=jnp.float32)
    su = jnp.dot(x32, wsu_ref[...], preferred_element_type=jnp.float32)
    inter_s = sg * jax.nn.sigmoid(sg) * su
    acc = jnp.dot(inter_s, wsd_ref[...], preferred_element_type=jnp.float32)

    for e in range(E):
        g = jnp.dot(x32, wg_ref[e], preferred_element_type=jnp.float32)
        u = jnp.dot(x32, wu_ref[e], preferred_element_type=jnp.float32)
        he_in = c[:, e:e + 1] * (g * jax.nn.sigmoid(g) * u)
        acc = acc + jnp.dot(he_in, wd_ref[e],
                            preferred_element_type=jnp.float32)

    out_ref[...] = acc


@functools.partial(jax.jit, static_argnames=("bt",))
def _moe(x, gwt, wg, wu, wd, wsg, wsu, wsd, bt=256):
    grid = (T // bt,)
    return pl.pallas_call(
        _moe_block,
        grid=grid,
        in_specs=[
            pl.BlockSpec((bt, H), lambda i: (i, 0)),
            pl.BlockSpec((H, E), lambda i: (0, 0)),
            pl.BlockSpec((E, H, F), lambda i: (0, 0, 0)),
            pl.BlockSpec((E, H, F), lambda i: (0, 0, 0)),
            pl.BlockSpec((E, F, H), lambda i: (0, 0, 0)),
            pl.BlockSpec((H, SHARED_F), lambda i: (0, 0)),
            pl.BlockSpec((H, SHARED_F), lambda i: (0, 0)),
            pl.BlockSpec((SHARED_F, H), lambda i: (0, 0)),
        ],
        out_specs=pl.BlockSpec((bt, H), lambda i: (i, 0)),
        out_shape=jax.ShapeDtypeStruct((T, H), jnp.float32),
    )(x, gwt, wg, wu, wd, wsg, wsu, wsd)


def kernel(hidden_states, gate_w, w_gate, w_up, w_down, ws_gate, ws_up,
           ws_down):
    x = hidden_states.reshape(-1, H)
    out = _moe(x, gate_w.T, w_gate, w_up, w_down, ws_gate, ws_up, ws_down)
    return out.reshape(B, S, H)
